# baseline (speedup 1.0000x reference)
import functools

import jax
import jax.numpy as jnp
from jax import lax
from jax.experimental import pallas as pl
from jax.experimental.pallas import tpu as pltpu
from jax.experimental.pallas import tpu_sc as plsc

NC = 2
NS = 16
NW = NC * NS
D = 32
HALF = 16
CHUNK = 128

_MESH = plsc.VectorSubcoreMesh(core_axis_name="c", subcore_axis_name="s")
_NO_TC_TILING = pltpu.CompilerParams(use_tc_tiling_on_sc=False)


def _sc_gather(P2, Q2, ub, ib, uidx2, iidx2, uid, iid):
    B = uid.shape[0]
    b_per_w = B // NW
    r_per_w = 2 * b_per_w
    n_ch2 = r_per_w // CHUNK
    n_ch = b_per_w // CHUNK

    @functools.partial(
        pl.kernel,
        mesh=_MESH,
        compiler_params=_NO_TC_TILING,
        out_type=(
            jax.ShapeDtypeStruct((2 * B, HALF), jnp.float32),
            jax.ShapeDtypeStruct((2 * B, HALF), jnp.float32),
            jax.ShapeDtypeStruct((B,), jnp.float32),
            jax.ShapeDtypeStruct((B,), jnp.float32),
        ),
        scratch_types=[
            pltpu.VMEM((r_per_w,), jnp.int32),
            pltpu.VMEM((r_per_w,), jnp.int32),
            pltpu.VMEM((b_per_w,), jnp.int32),
            pltpu.VMEM((b_per_w,), jnp.int32),
            pltpu.VMEM((r_per_w, HALF), jnp.float32),
            pltpu.VMEM((r_per_w, HALF), jnp.float32),
            pltpu.VMEM((b_per_w,), jnp.float32),
            pltpu.VMEM((b_per_w,), jnp.float32),
            pltpu.SemaphoreType.DMA,
            pltpu.SemaphoreType.DMA,
        ],
    )
    def k(P_hbm, Q_hbm, ub_hbm, ib_hbm, u2_hbm, i2_hbm, uid_hbm, iid_hbm,
          pu_out, qi_out, bu_out, bi_out,
          u2_v, i2_v, uid_v, iid_v, pr_v, qr_v, bu_v, bi_v, sem, sem2):
        wid = lax.axis_index("s") * NC + lax.axis_index("c")
        base = wid * b_per_w
        base2 = wid * r_per_w
        pltpu.sync_copy(u2_hbm.at[pl.ds(base2, r_per_w)], u2_v)
        pltpu.sync_copy(i2_hbm.at[pl.ds(base2, r_per_w)], i2_v)
        pltpu.sync_copy(uid_hbm.at[pl.ds(base, b_per_w)], uid_v)
        pltpu.sync_copy(iid_hbm.at[pl.ds(base, b_per_w)], iid_v)
        gathers = []
        for c in range(n_ch2):
            sl = pl.ds(c * CHUNK, CHUNK)
            gathers.append(pltpu.async_copy(P_hbm.at[u2_v.at[sl]], pr_v.at[sl], sem))
            gathers.append(pltpu.async_copy(Q_hbm.at[i2_v.at[sl]], qr_v.at[sl], sem))
        for c in range(n_ch):
            sl = pl.ds(c * CHUNK, CHUNK)
            gathers.append(pltpu.async_copy(ub_hbm.at[uid_v.at[sl]], bu_v.at[sl], sem))
            gathers.append(pltpu.async_copy(ib_hbm.at[iid_v.at[sl]], bi_v.at[sl], sem))
        for g in gathers:
            g.wait()
        outs = [
            pltpu.async_copy(pr_v, pu_out.at[pl.ds(base2, r_per_w)], sem2),
            pltpu.async_copy(qr_v, qi_out.at[pl.ds(base2, r_per_w)], sem2),
            pltpu.async_copy(bu_v, bu_out.at[pl.ds(base, b_per_w)], sem2),
            pltpu.async_copy(bi_v, bi_out.at[pl.ds(base, b_per_w)], sem2),
        ]
        for o in outs:
            o.wait()

    return k(P2, Q2, ub, ib, uidx2, iidx2, uid, iid)


def _reduce_body(p_ref, q_ref, bu_ref, bi_ref, o_ref):
    o_ref[...] = (jnp.sum(p_ref[...] * q_ref[...], axis=1)
                  + bu_ref[...] + bi_ref[...])


def _tc_reduce(pu, qi, bu, bi):
    B = pu.shape[0]
    nb = 8
    bb = B // nb
    return pl.pallas_call(
        _reduce_body,
        grid=(nb,),
        in_specs=[
            pl.BlockSpec((bb, D), lambda i: (i, 0)),
            pl.BlockSpec((bb, D), lambda i: (i, 0)),
            pl.BlockSpec((bb,), lambda i: (i,)),
            pl.BlockSpec((bb,), lambda i: (i,)),
        ],
        out_specs=pl.BlockSpec((bb,), lambda i: (i,)),
        out_shape=jax.ShapeDtypeStruct((B,), jnp.float32),
    )(pu, qi, bu, bi)


def kernel(user_id, item_id, P, Q, user_bias, item_bias):
    B = user_id.shape[0]
    P2 = P.reshape(2 * P.shape[0], HALF)
    Q2 = Q.reshape(2 * Q.shape[0], HALF)
    ub = user_bias.reshape(-1)
    ib = item_bias.reshape(-1)
    uidx2 = (2 * user_id[:, None] + jnp.arange(2, dtype=jnp.int32)).reshape(-1)
    iidx2 = (2 * item_id[:, None] + jnp.arange(2, dtype=jnp.int32)).reshape(-1)
    pu2, qi2, bu, bi = _sc_gather(P2, Q2, ub, ib, uidx2, iidx2, user_id, item_id)
    return _tc_reduce(pu2.reshape(B, D), qi2.reshape(B, D), bu, bi)
